# baseline (speedup 1.0000x reference)
import jax
import jax.numpy as jnp
from jax import lax
from jax.experimental import pallas as pl
from jax.experimental.pallas import tpu as pltpu
from jax.experimental.pallas import tpu_sc as plsc

_B, _S, _D = 4, 8192, 1024
_NC, _NS = 2, 16
_NW = _NC * _NS
_ROWS_PER_W = _S // _NW
_CH = 8
_NCH = _ROWS_PER_W // _CH


def _sc_body(x_hbm, pe_hbm, out_hbm, *refs):
    xin = [[refs[2 * b + p] for p in range(2)] for b in range(_B)]
    ob = list(refs[8:12])
    pev = list(refs[12:14])
    ld = [[refs[14 + 2 * b + p] for p in range(2)] for b in range(_B)]
    st = list(refs[22:26])
    ps = list(refs[26:28])

    wid = lax.axis_index("s") * _NC + lax.axis_index("c")
    base = wid * _ROWS_PER_W

    for c in range(2):
        pltpu.async_copy(pe_hbm.at[pl.ds(base + c * _CH, _CH)], pev[c], ps[c])
        for b in range(_B):
            pltpu.async_copy(
                x_hbm.at[pl.ds(base + b * _S + c * _CH, _CH)], xin[b][c],
                ld[b][c])
    for b in range(_B):
        pltpu.async_copy(ob[b], out_hbm.at[pl.ds(base + b * _S, _CH)], st[b])

    def outer(k, carry):
        ci0 = 2 * k
        for p in range(2):
            ci = ci0 + p
            row = base + ci * _CH
            pltpu.make_async_copy(
                pe_hbm.at[pl.ds(0, _CH)], pev[p], ps[p]).wait()
            for b in range(_B):
                pltpu.make_async_copy(
                    x_hbm.at[pl.ds(0, _CH)], xin[b][p], ld[b][p]).wait()
                pltpu.make_async_copy(
                    ob[b], out_hbm.at[pl.ds(0, _CH)], st[b]).wait()

            xin_p = [xin[b][p] for b in range(_B)]
            pev_p = pev[p]

            for r in range(_CH):
                @plsc.parallel_loop(0, _D, 16, unroll=8)
                def add_body(c, _r=r):
                    pe16 = pev_p[_r, pl.ds(c, 16)]
                    for b in range(_B):
                        ob[b][_r, pl.ds(c, 16)] = (
                            xin_p[b][_r, pl.ds(c, 16)] + pe16)

            for b in range(_B):
                pltpu.async_copy(
                    ob[b], out_hbm.at[pl.ds(row + b * _S, _CH)], st[b])

            @pl.when(ci < _NCH - 2)
            def _():
                pltpu.async_copy(
                    pe_hbm.at[pl.ds(row + 2 * _CH, _CH)], pev[p], ps[p])
                for b in range(_B):
                    pltpu.async_copy(
                        x_hbm.at[pl.ds(row + b * _S + 2 * _CH, _CH)],
                        xin[b][p], ld[b][p])
        return carry

    lax.fori_loop(0, _NCH // 2, outer, 0)

    for b in range(_B):
        pltpu.make_async_copy(ob[b], out_hbm.at[pl.ds(0, _CH)], st[b]).wait()


def kernel(x, pe):
    B, S, D = x.shape
    mesh = plsc.VectorSubcoreMesh(core_axis_name="c", subcore_axis_name="s")
    out2d = pl.kernel(
        _sc_body,
        out_type=jax.ShapeDtypeStruct((B * S, D), jnp.float32),
        mesh=mesh,
        scratch_types=(
            [pltpu.VMEM((_CH, _D), jnp.float32) for _ in range(14)]
            + [pltpu.SemaphoreType.DMA for _ in range(14)]
        ),
        compiler_params=pltpu.CompilerParams(use_tc_tiling_on_sc=True),
    )(x.reshape(B * S, D), pe)
    return out2d.reshape(B, S, D)
